# baseline (speedup 1.0000x reference)
import functools

import jax
import jax.numpy as jnp
from jax import lax
from jax.experimental import pallas as pl
from jax.experimental.pallas import tpu as pltpu
from jax.experimental.pallas import tpu_sc as plsc

N_NODES = 10000
N_EDGES = 160000
D_IN = 256
D_OUT = 256
H = 128
NC = 2
NS = 16
EC = 64
CHUNKS_PER_SUB = 168
MB = 24
MACROS = CHUNKS_PER_SUB // MB
E_PAD = CHUNKS_PER_SUB * NS * EC
N_PAD = 10112
ROWS_PER_SUB = N_PAD // NS
CROWS = 80


def _sc_segment_sum(x2, srcs4, dst3, oh, zrs):
    mesh = plsc.VectorSubcoreMesh(
        core_axis_name="c", subcore_axis_name="s",
        num_cores=NC, num_subcores=NS)

    @functools.partial(
        pl.kernel,
        mesh=mesh,
        out_type=(
            jax.ShapeDtypeStruct((NC, N_PAD, H), jnp.float32),
            jax.ShapeDtypeStruct((NC, CROWS, 128), jnp.float32),
        ),
        scratch_types=[
            pltpu.VMEM((2, 8, EC), jnp.int32),
            pltpu.VMEM((2, 8, EC), jnp.int32),
            pltpu.VMEM((EC, H), jnp.float32),
            pltpu.VMEM((EC, H), jnp.float32),
            pltpu.VMEM((EC, H), jnp.float32),
            pltpu.VMEM((EC, 128), jnp.float32),
            pltpu.VMEM((EC, 128), jnp.float32),
            pltpu.VMEM((EC,), jnp.int32),
            pltpu.VMEM((2, EC), jnp.int32),
            pltpu.VMEM_SHARED((N_PAD, H), jnp.float32),
            pltpu.VMEM_SHARED((CROWS, 128), jnp.float32),
            pltpu.SemaphoreType.DMA,
            pltpu.SemaphoreType.DMA,
            pltpu.SemaphoreType.DMA,
            pltpu.SemaphoreType.DMA,
            pltpu.SemaphoreType.DMA,
            pltpu.SemaphoreType.DMA,
            pltpu.SemaphoreType.DMA,
            pltpu.SemaphoreType.DMA,
            pltpu.SemaphoreType.DMA,
            pltpu.SemaphoreType.DMA,
            pltpu.SemaphoreType.DMA,
        ],
    )
    def sc_kernel(x2_hbm, srcs_hbm, dst_hbm, oh_hbm, zrs_hbm, s_hbm, cnt_hbm,
                  sidx, didx, r0, r1, r2, ca, cb, cl, ch, acc, cnt,
                  g0, g1, g2, s0, s1, s2, cga, cgb, csa, csb, sem_i):
        cid = lax.axis_index("c")
        sid = lax.axis_index("s")
        rows = (r0, r1, r2)
        gsem = (g0, g1, g2)
        ssem = (s0, s1, s2)
        cbuf = (ca, cb)
        cgsem = (cga, cgb)
        cssem = (csa, csb)

        base = sid * ROWS_PER_SUB
        pltpu.sync_copy(zrs_hbm.at[pl.ds(base, ROWS_PER_SUB)],
                        acc.at[pl.ds(base, ROWS_PER_SUB)])

        @pl.when(sid < CROWS // 8)
        def _():
            pltpu.sync_copy(zrs_hbm.at[pl.ds(sid * 8, 8)],
                            cnt.at[pl.ds(sid * 8, 8)])

        plsc.subcore_barrier()

        def issue_gather(idx_ref, buf, sem):
            pltpu.async_copy(x2_hbm.at[idx_ref], buf, sem)

        def wait_gather(buf, sem):
            pltpu.make_async_copy(x2_hbm.at[cl], buf, sem).wait()

        def issue_scatter(buf, idx_ref, sem):
            pltpu.async_copy(buf, acc.at[idx_ref], sem, add=True)

        def wait_scatter(buf, sem):
            pltpu.make_async_copy(buf, acc.at[pl.ds(0, EC)], sem).wait()

        def count_begin(kb, j, m):
            for u in range(EC // 16):
                d16 = didx[kb, j, pl.ds(u * 16, 16)]
                cl[pl.ds(u * 16, 16)] = lax.bitwise_and(d16, 127)
                ch[m, pl.ds(u * 16, 16)] = lax.shift_right_logical(d16, 7)
            pltpu.async_copy(oh_hbm.at[cl], cbuf[m], cgsem[m])

        def count_finish(m):
            pltpu.make_async_copy(oh_hbm.at[cl], cbuf[m], cgsem[m]).wait()
            pltpu.async_copy(cbuf[m], cnt.at[ch.at[m]], cssem[m], add=True)

        def count_drain(m):
            pltpu.make_async_copy(cbuf[m], cnt.at[pl.ds(0, EC)],
                                  cssem[m]).wait()

        pltpu.sync_copy(srcs_hbm.at[cid, sid, 0], sidx.at[0])
        pltpu.sync_copy(dst_hbm.at[sid, 0], didx.at[0])
        issue_gather(sidx.at[0, 0], r0, g0)
        issue_gather(sidx.at[0, 1], r1, g1)

        def macro(kk, _):
            kb = lax.rem(kk, 2)

            for j in range(MB):
                jp2 = (j + 2) % 3
                par = j % 2
                m = (0, 0, 1, 1, 0, 0)[j % 6]
                fin_m = (0, 0, 0, 1, 1, 0)[j % 6]

                if j > 0:
                    wait_scatter(rows[jp2], ssem[jp2])
                else:
                    @pl.when(kk > 0)
                    def _(jp2=jp2):
                        wait_scatter(rows[jp2], ssem[jp2])

                if j % 8 == 0:
                    i1 = j // 8 + 1
                    def _pref(i1=i1):
                        nb = 3 * kk + i1
                        ns_ = lax.rem(kk + i1, 2)
                        pltpu.async_copy(srcs_hbm.at[cid, sid, nb],
                                         sidx.at[ns_], sem_i)
                        pltpu.async_copy(dst_hbm.at[sid, nb],
                                         didx.at[ns_], sem_i)
                    if j // 8 < 2:
                        _pref()
                    else:
                        pl.when(kk < MACROS - 1)(_pref)

                if j == 0:
                    @pl.when(jnp.logical_and(cid == 1 - par, kk > 0))
                    def _(fin_m=fin_m):
                        count_finish(fin_m)
                else:
                    @pl.when(cid == 1 - par)
                    def _(fin_m=fin_m):
                        count_finish(fin_m)

                if j < 4:
                    @pl.when(jnp.logical_and(cid == par, kk > 0))
                    def _(m=m):
                        count_drain(m)
                else:
                    @pl.when(cid == par)
                    def _(m=m):
                        count_drain(m)

                @pl.when(cid == par)
                def _(j=j, m=m):
                    count_begin(lax.rem(kk + j // 8, 2), j % 8, m)

                if j % 8 == 6:
                    def _waiti():
                        pltpu.make_async_copy(srcs_hbm.at[cid, sid, 0],
                                              sidx.at[0], sem_i).wait()
                        pltpu.make_async_copy(dst_hbm.at[sid, 0],
                                              didx.at[0], sem_i).wait()
                    if j // 8 < 2:
                        _waiti()
                    else:
                        pl.when(kk < MACROS - 1)(_waiti)

                sb_g = lax.rem(kk + (j + 2) // 8, 2)
                if j < MB - 2:
                    issue_gather(sidx.at[sb_g, (j + 2) % 8], rows[jp2],
                                 gsem[jp2])
                else:
                    @pl.when(kk < MACROS - 1)
                    def _(j=j, jp2=jp2, sb_g=sb_g):
                        issue_gather(sidx.at[sb_g, (j + 2) % 8], rows[jp2],
                                     gsem[jp2])

                sb_c = lax.rem(kk + j // 8, 2)
                wait_gather(rows[j % 3], gsem[j % 3])
                issue_scatter(rows[j % 3], didx.at[sb_c, j % 8],
                              ssem[j % 3])

            return 0

        lax.fori_loop(0, MACROS, macro, 0)

        @pl.when(cid == 1)
        def _():
            count_finish(0)

        wait_scatter(rows[2], ssem[2])
        count_drain(0)
        count_drain(1)

        plsc.subcore_barrier()

        pltpu.sync_copy(acc.at[pl.ds(base, ROWS_PER_SUB)],
                        s_hbm.at[cid, pl.ds(base, ROWS_PER_SUB)])

        @pl.when(sid < CROWS // 8)
        def _():
            pltpu.sync_copy(cnt.at[pl.ds(sid * 8, 8)],
                            cnt_hbm.at[cid, pl.ds(sid * 8, 8)])

    return sc_kernel(x2, srcs4, dst3, oh, zrs)


BR = 1000


def _tc_finish_body(x_ref, s0_ref, s1_ref, c0_ref, c1_ref, w_ref, b_ref,
                    o_ref):
    s = jnp.concatenate([s0_ref[0], s1_ref[0]], axis=1)
    c = jnp.maximum(c0_ref[...] + c1_ref[...], 1.0)
    h = s / c + x_ref[...]
    o_ref[...] = (
        jnp.dot(h, w_ref[...], preferred_element_type=jnp.float32,
                precision=lax.Precision.HIGHEST) + b_ref[...]
    )


def _tc_finish(x, s, c0, c1, W, b2):
    grid = (N_NODES // BR,)
    return pl.pallas_call(
        _tc_finish_body,
        grid=grid,
        in_specs=[
            pl.BlockSpec((BR, D_IN), lambda i: (i, 0)),
            pl.BlockSpec((1, BR, H), lambda i: (0, i, 0)),
            pl.BlockSpec((1, BR, H), lambda i: (1, i, 0)),
            pl.BlockSpec((BR, 1), lambda i: (i, 0)),
            pl.BlockSpec((BR, 1), lambda i: (i, 0)),
            pl.BlockSpec((D_IN, D_OUT), lambda i: (0, 0)),
            pl.BlockSpec((1, D_OUT), lambda i: (0, 0)),
        ],
        out_specs=pl.BlockSpec((BR, D_OUT), lambda i: (i, 0)),
        out_shape=jax.ShapeDtypeStruct((N_NODES, D_OUT), jnp.float32),
    )(x, s, s, c0, c1, W, b2)


def kernel(x, edge_index, W, b):
    src = edge_index[0].astype(jnp.int32)
    dst = edge_index[1].astype(jnp.int32)
    pad = E_PAD - N_EDGES
    srcs = jnp.concatenate([
        src, jnp.zeros((pad,), jnp.int32),
        src + N_NODES, jnp.full((pad,), N_NODES, jnp.int32),
    ]).reshape(NC, NS, CHUNKS_PER_SUB // 8, 8, EC)
    dst_p = jnp.concatenate(
        [dst, jnp.full((pad,), N_NODES, jnp.int32)]
    ).reshape(NS, CHUNKS_PER_SUB // 8, 8, EC)
    x2 = jnp.concatenate([x[:, :H], x[:, H:]], axis=0)
    oh = jnp.eye(128, dtype=jnp.float32)
    zrs = jnp.zeros((N_PAD, H), jnp.float32)

    s, cnt = _sc_segment_sum(x2, srcs, dst_p, oh, zrs)
    c0 = cnt[0].reshape(CROWS * 128, 1)[:N_NODES]
    c1 = cnt[1].reshape(CROWS * 128, 1)[:N_NODES]
    return _tc_finish(x, s, c0, c1, W, b.reshape(1, D_OUT))


# speedup vs baseline: 1.4043x; 1.4043x over previous
import functools

import jax
import jax.numpy as jnp
from jax import lax
from jax.experimental import pallas as pl
from jax.experimental.pallas import tpu as pltpu
from jax.experimental.pallas import tpu_sc as plsc

N_NODES = 10000
N_EDGES = 160000
D_IN = 256
D_OUT = 256
H = 128
NC = 2
NS = 16
EC = 256
CHUNKS_PER_SUB = 40
E_PAD = CHUNKS_PER_SUB * NS * EC
N_PAD = 10112
ROWS_PER_SUB = N_PAD // NS
CROWS = 80


def _sc_segment_sum(x2, srcs5, dst4, oh, zrs):
    mesh = plsc.VectorSubcoreMesh(
        core_axis_name="c", subcore_axis_name="s",
        num_cores=NC, num_subcores=NS)

    @functools.partial(
        pl.kernel,
        mesh=mesh,
        out_type=(
            jax.ShapeDtypeStruct((NC, N_PAD, H), jnp.float32),
            jax.ShapeDtypeStruct((NC, CROWS, 128), jnp.float32),
        ),
        scratch_types=[
            pltpu.VMEM((8 * EC,), jnp.int32),
            pltpu.VMEM((CHUNKS_PER_SUB * EC,), jnp.int32),
            pltpu.VMEM((EC, H), jnp.float32),
            pltpu.VMEM((EC,), jnp.int32),
            pltpu.VMEM((EC,), jnp.int32),
            pltpu.VMEM_SHARED((N_PAD, H), jnp.float32),
            pltpu.VMEM_SHARED((CROWS, 128), jnp.float32),
            pltpu.SemaphoreType.DMA,
        ],
    )
    def sc_kernel(x2_hbm, srcs_hbm, dst_hbm, oh_hbm, zrs_hbm, s_hbm, cnt_hbm,
                  sidx, didx, rows, cl, ch, acc, cnt, sem):
        cid = lax.axis_index("c")
        sid = lax.axis_index("s")

        base = sid * ROWS_PER_SUB
        pltpu.sync_copy(zrs_hbm.at[pl.ds(base, ROWS_PER_SUB)],
                        acc.at[pl.ds(base, ROWS_PER_SUB)])

        @pl.when(sid < CROWS // 8)
        def _():
            pltpu.sync_copy(zrs_hbm.at[pl.ds(sid * 8, 8)],
                            cnt.at[pl.ds(sid * 8, 8)])

        pltpu.sync_copy(dst_hbm.at[sid], didx)

        plsc.subcore_barrier()

        def step(c, _):
            @pl.when(lax.rem(c, 8) == 0)
            def _():
                pltpu.sync_copy(srcs_hbm.at[cid, sid, lax.div(c, 8)], sidx)

            pltpu.async_copy(
                x2_hbm.at[sidx.at[pl.ds(lax.rem(c, 8) * EC, EC)]], rows,
                sem).wait()
            pltpu.sync_copy(rows, acc.at[didx.at[pl.ds(c * EC, EC)]],
                            add=True)

            @pl.when(lax.rem(c, 2) == cid)
            def _():
                for v in range(EC // 16):
                    d16 = didx[pl.ds(c * EC + v * 16, 16)]
                    cl[pl.ds(v * 16, 16)] = lax.bitwise_and(d16, 127)
                    ch[pl.ds(v * 16, 16)] = lax.shift_right_logical(d16, 7)
                pltpu.async_copy(oh_hbm.at[cl], rows, sem).wait()
                pltpu.sync_copy(rows, cnt.at[ch], add=True)

            return 0

        lax.fori_loop(0, CHUNKS_PER_SUB, step, 0)

        plsc.subcore_barrier()

        pltpu.sync_copy(acc.at[pl.ds(base, ROWS_PER_SUB)],
                        s_hbm.at[cid, pl.ds(base, ROWS_PER_SUB)])

        @pl.when(sid < CROWS // 8)
        def _():
            pltpu.sync_copy(cnt.at[pl.ds(sid * 8, 8)],
                            cnt_hbm.at[cid, pl.ds(sid * 8, 8)])

    return sc_kernel(x2, srcs5, dst4, oh, zrs)


BR = 1000


def _tc_finish_body(x_ref, s0_ref, s1_ref, c0_ref, c1_ref, w_ref, b_ref,
                    o_ref):
    s = jnp.concatenate([s0_ref[0], s1_ref[0]], axis=1)
    c = jnp.maximum(c0_ref[...] + c1_ref[...], 1.0)
    h = s / c + x_ref[...]
    o_ref[...] = (
        jnp.dot(h, w_ref[...], preferred_element_type=jnp.float32,
                precision=lax.Precision.HIGHEST) + b_ref[...]
    )


def _tc_finish(x, s, c0, c1, W, b2):
    grid = (N_NODES // BR,)
    return pl.pallas_call(
        _tc_finish_body,
        grid=grid,
        in_specs=[
            pl.BlockSpec((BR, D_IN), lambda i: (i, 0)),
            pl.BlockSpec((1, BR, H), lambda i: (0, i, 0)),
            pl.BlockSpec((1, BR, H), lambda i: (1, i, 0)),
            pl.BlockSpec((BR, 1), lambda i: (i, 0)),
            pl.BlockSpec((BR, 1), lambda i: (i, 0)),
            pl.BlockSpec((D_IN, D_OUT), lambda i: (0, 0)),
            pl.BlockSpec((1, D_OUT), lambda i: (0, 0)),
        ],
        out_specs=pl.BlockSpec((BR, D_OUT), lambda i: (i, 0)),
        out_shape=jax.ShapeDtypeStruct((N_NODES, D_OUT), jnp.float32),
    )(x, s, s, c0, c1, W, b2)


def kernel(x, edge_index, W, b):
    src = edge_index[0].astype(jnp.int32)
    dst = edge_index[1].astype(jnp.int32)
    pad = E_PAD - N_EDGES
    srcs = jnp.concatenate([
        src, jnp.zeros((pad,), jnp.int32),
        src + N_NODES, jnp.full((pad,), N_NODES, jnp.int32),
    ]).reshape(NC, NS, CHUNKS_PER_SUB // 8, 8 * EC)
    dst_p = jnp.concatenate(
        [dst, jnp.full((pad,), N_NODES, jnp.int32)]
    ).reshape(NS, CHUNKS_PER_SUB * EC)
    x2 = jnp.concatenate([x[:, :H], x[:, H:]], axis=0)
    oh = jnp.eye(128, dtype=jnp.float32)
    zrs = jnp.zeros((N_PAD, H), jnp.float32)

    s, cnt = _sc_segment_sum(x2, srcs, dst_p, oh, zrs)
    c0 = cnt[0].reshape(CROWS * 128, 1)[:N_NODES]
    c1 = cnt[1].reshape(CROWS * 128, 1)[:N_NODES]
    return _tc_finish(x, s, c0, c1, W, b.reshape(1, D_OUT))


# speedup vs baseline: 1.4718x; 1.0480x over previous
import functools

import jax
import jax.numpy as jnp
from jax import lax
from jax.experimental import pallas as pl
from jax.experimental.pallas import tpu as pltpu
from jax.experimental.pallas import tpu_sc as plsc

N_NODES = 10000
N_EDGES = 160000
D_IN = 256
D_OUT = 256
H = 128
NC = 2
NS = 16
EC = 128
CHUNKS_PER_SUB = 80
E_PAD = CHUNKS_PER_SUB * NS * EC
N_PAD = 10112
ROWS_PER_SUB = N_PAD // NS
ZROWS = 8
CROWS = 80


def _sc_segment_sum(x2, srcs4, dst3, oh, zrs):
    mesh = plsc.VectorSubcoreMesh(
        core_axis_name="c", subcore_axis_name="s",
        num_cores=NC, num_subcores=NS)

    @functools.partial(
        pl.kernel,
        mesh=mesh,
        out_type=(
            jax.ShapeDtypeStruct((NC, N_PAD, H), jnp.float32),
            jax.ShapeDtypeStruct((NC, CROWS, 128), jnp.float32),
        ),
        scratch_types=[
            pltpu.VMEM((CHUNKS_PER_SUB, EC), jnp.int32),
            pltpu.VMEM((2, 2, EC), jnp.int32),
            pltpu.VMEM((EC,), jnp.int32),
            pltpu.VMEM((EC,), jnp.int32),
            pltpu.VMEM((EC, H), jnp.float32),
            pltpu.VMEM((EC, H), jnp.float32),
            pltpu.VMEM_SHARED((N_PAD, H), jnp.float32),
            pltpu.VMEM_SHARED((CROWS, 128), jnp.float32),
            pltpu.SemaphoreType.DMA,
            pltpu.SemaphoreType.DMA,
            pltpu.SemaphoreType.DMA,
            pltpu.SemaphoreType.DMA,
            pltpu.SemaphoreType.DMA,
        ],
    )
    def sc_kernel(x2_hbm, srcs_hbm, dst_hbm, oh_hbm, zrs_hbm, s_hbm,
                  cnt_hbm, sidx, didx, cl, ch, rows_a, rows_b, acc, cnt,
                  sem_a, sem_b, sem_i, sem_sa, sem_sb):
        cid = lax.axis_index("c")
        sid = lax.axis_index("s")
        base = sid * ROWS_PER_SUB
        pltpu.sync_copy(zrs_hbm.at[pl.ds(base, ROWS_PER_SUB)],
                        acc.at[pl.ds(base, ROWS_PER_SUB)])

        @pl.when(sid < CROWS // ZROWS)
        def _():
            pltpu.sync_copy(zrs_hbm.at[pl.ds(sid * ZROWS, ZROWS)],
                            cnt.at[pl.ds(sid * ZROWS, ZROWS)])

        pltpu.sync_copy(srcs_hbm.at[cid, sid], sidx)
        pltpu.async_copy(dst_hbm.at[sid, pl.ds(0, 2)], didx.at[0], sem_i)

        plsc.subcore_barrier()

        def count_phase(kb, r, buf, sem):
            for j in range(EC // 16):
                d16 = didx[kb, r, pl.ds(j * 16, 16)]
                cl[pl.ds(j * 16, 16)] = lax.bitwise_and(d16, 127)
                ch[pl.ds(j * 16, 16)] = lax.shift_right_logical(d16, 7)
            pltpu.async_copy(oh_hbm.at[cl], buf, sem).wait()
            pltpu.sync_copy(buf, cnt.at[ch], add=True)

        pltpu.async_copy(x2_hbm.at[sidx.at[0]], rows_a, sem_a)

        def step(k, _):
            c0 = 2 * k
            kb = lax.rem(k, 2)
            pltpu.make_async_copy(
                dst_hbm.at[sid, pl.ds(0, 2)], didx.at[kb], sem_i).wait()

            @pl.when(k < CHUNKS_PER_SUB // 2 - 1)
            def _():
                pltpu.async_copy(dst_hbm.at[sid, pl.ds(c0 + 2, 2)],
                                 didx.at[1 - kb], sem_i)

            pltpu.async_copy(x2_hbm.at[sidx.at[c0 + 1]], rows_b, sem_b)
            pltpu.make_async_copy(x2_hbm.at[sidx.at[c0]], rows_a, sem_a).wait()
            pltpu.async_copy(rows_a, acc.at[didx.at[kb, 0]], sem_sa, add=True)
            pltpu.make_async_copy(
                x2_hbm.at[sidx.at[c0 + 1]], rows_b, sem_b).wait()
            pltpu.async_copy(rows_b, acc.at[didx.at[kb, 1]], sem_sb, add=True)

            pltpu.make_async_copy(rows_a, acc.at[pl.ds(0, EC)], sem_sa).wait()

            @pl.when(cid == 0)
            def _():
                count_phase(kb, 0, rows_a, sem_a)

            @pl.when(k < CHUNKS_PER_SUB // 2 - 1)
            def _():
                pltpu.async_copy(x2_hbm.at[sidx.at[c0 + 2]], rows_a, sem_a)

            pltpu.make_async_copy(rows_b, acc.at[pl.ds(0, EC)], sem_sb).wait()

            @pl.when(cid == 1)
            def _():
                count_phase(kb, 1, rows_b, sem_b)

            return 0

        lax.fori_loop(0, CHUNKS_PER_SUB // 2, step, 0)

        plsc.subcore_barrier()

        pltpu.sync_copy(acc.at[pl.ds(base, ROWS_PER_SUB)],
                        s_hbm.at[cid, pl.ds(base, ROWS_PER_SUB)])

        @pl.when(sid < CROWS // ZROWS)
        def _():
            pltpu.sync_copy(cnt.at[pl.ds(sid * ZROWS, ZROWS)],
                            cnt_hbm.at[cid, pl.ds(sid * ZROWS, ZROWS)])

    return sc_kernel(x2, srcs4, dst3, oh, zrs)


BR = 1000


def _tc_finish_body(x_ref, s0_ref, s1_ref, c0_ref, c1_ref, w_ref, b_ref,
                    o_ref):
    s = jnp.concatenate([s0_ref[0], s1_ref[0]], axis=1)
    c = jnp.maximum(c0_ref[...] + c1_ref[...], 1.0)
    h = s / c + x_ref[...]
    o_ref[...] = (
        jnp.dot(h, w_ref[...], preferred_element_type=jnp.float32,
                precision=lax.Precision.HIGHEST) + b_ref[...]
    )


def _tc_finish(x, s, c0, c1, W, b2):
    grid = (N_NODES // BR,)
    return pl.pallas_call(
        _tc_finish_body,
        grid=grid,
        in_specs=[
            pl.BlockSpec((BR, D_IN), lambda i: (i, 0)),
            pl.BlockSpec((1, BR, H), lambda i: (0, i, 0)),
            pl.BlockSpec((1, BR, H), lambda i: (1, i, 0)),
            pl.BlockSpec((BR, 1), lambda i: (i, 0)),
            pl.BlockSpec((BR, 1), lambda i: (i, 0)),
            pl.BlockSpec((D_IN, D_OUT), lambda i: (0, 0)),
            pl.BlockSpec((1, D_OUT), lambda i: (0, 0)),
        ],
        out_specs=pl.BlockSpec((BR, D_OUT), lambda i: (i, 0)),
        out_shape=jax.ShapeDtypeStruct((N_NODES, D_OUT), jnp.float32),
    )(x, s, s, c0, c1, W, b2)


def kernel(x, edge_index, W, b):
    src = edge_index[0].astype(jnp.int32)
    dst = edge_index[1].astype(jnp.int32)
    pad = E_PAD - N_EDGES
    srcs = jnp.concatenate([
        src, jnp.zeros((pad,), jnp.int32),
        src + N_NODES, jnp.full((pad,), N_NODES, jnp.int32),
    ]).reshape(NC, NS, CHUNKS_PER_SUB, EC)
    dst_p = jnp.concatenate(
        [dst, jnp.full((pad,), N_NODES, jnp.int32)]
    ).reshape(NS, CHUNKS_PER_SUB, EC)
    x2 = jnp.concatenate([x[:, :H], x[:, H:]], axis=0)
    oh = jnp.eye(128, dtype=jnp.float32)
    zrs = jnp.zeros((N_PAD, H), jnp.float32)

    s, cnt = _sc_segment_sum(x2, srcs, dst_p, oh, zrs)
    c0 = cnt[0].reshape(CROWS * 128, 1)[:N_NODES]
    c1 = cnt[1].reshape(CROWS * 128, 1)[:N_NODES]
    return _tc_finish(x, s, c0, c1, W, b.reshape(1, D_OUT))


# speedup vs baseline: 1.8939x; 1.2867x over previous
import functools

import jax
import jax.numpy as jnp
from jax import lax
from jax.experimental import pallas as pl
from jax.experimental.pallas import tpu as pltpu
from jax.experimental.pallas import tpu_sc as plsc

N_NODES = 10000
N_EDGES = 160000
D_IN = 256
D_OUT = 256
H = 128
NC = 2
NS = 16
EC = 128
CHUNKS_PER_SUB = 80
E_PAD = CHUNKS_PER_SUB * NS * EC
N_PAD = 10112
ROWS_PER_SUB = N_PAD // NS
ZROWS = 8
CROWS = 80


def _sc_segment_sum(x2, srcs4, dst3, oh, zrs):
    mesh = plsc.VectorSubcoreMesh(
        core_axis_name="c", subcore_axis_name="s",
        num_cores=NC, num_subcores=NS)

    @functools.partial(
        pl.kernel,
        mesh=mesh,
        out_type=(
            jax.ShapeDtypeStruct((NC, N_PAD, H), jnp.float32),
            jax.ShapeDtypeStruct((NC, CROWS, 128), jnp.float32),
        ),
        scratch_types=[
            pltpu.VMEM((CHUNKS_PER_SUB, EC), jnp.int32),
            pltpu.VMEM((2, 2, EC), jnp.int32),
            pltpu.VMEM((EC,), jnp.int32),
            pltpu.VMEM((EC,), jnp.int32),
            pltpu.VMEM((EC, H), jnp.float32),
            pltpu.VMEM((EC, H), jnp.float32),
            pltpu.VMEM_SHARED((N_PAD, H), jnp.float32),
            pltpu.VMEM_SHARED((CROWS, 128), jnp.float32),
            pltpu.VMEM_SHARED((128, 128), jnp.float32),
            pltpu.SemaphoreType.DMA,
            pltpu.SemaphoreType.DMA,
            pltpu.SemaphoreType.DMA,
        ],
    )
    def sc_kernel(x2_hbm, srcs_hbm, dst_hbm, oh_hbm, zrs_hbm, s_hbm,
                  cnt_hbm, sidx, didx, cl, ch, rows_a, rows_b, acc, cnt,
                  oh_sh, sem_a, sem_b, sem_i):
        cid = lax.axis_index("c")
        sid = lax.axis_index("s")
        base = sid * ROWS_PER_SUB
        pltpu.sync_copy(zrs_hbm.at[pl.ds(base, ROWS_PER_SUB)],
                        acc.at[pl.ds(base, ROWS_PER_SUB)])

        @pl.when(sid < CROWS // ZROWS)
        def _():
            pltpu.sync_copy(zrs_hbm.at[pl.ds(sid * ZROWS, ZROWS)],
                            cnt.at[pl.ds(sid * ZROWS, ZROWS)])

        pltpu.sync_copy(oh_hbm.at[pl.ds(sid * 8, 8)],
                        oh_sh.at[pl.ds(sid * 8, 8)])

        pltpu.sync_copy(srcs_hbm.at[cid, sid], sidx)
        pltpu.async_copy(dst_hbm.at[sid, pl.ds(0, 2)], didx.at[0], sem_i)

        plsc.subcore_barrier()

        def count_phase(kb, r, buf, sem):
            for j in range(EC // 16):
                d16 = didx[kb, r, pl.ds(j * 16, 16)]
                cl[pl.ds(j * 16, 16)] = lax.bitwise_and(d16, 127)
                ch[pl.ds(j * 16, 16)] = lax.shift_right_logical(d16, 7)
            pltpu.async_copy(oh_sh.at[cl], buf, sem).wait()
            pltpu.sync_copy(buf, cnt.at[ch], add=True)

        pltpu.async_copy(x2_hbm.at[sidx.at[0]], rows_a, sem_a)

        def step(k, _):
            c0 = 2 * k
            kb = lax.rem(k, 2)
            pltpu.make_async_copy(
                dst_hbm.at[sid, pl.ds(0, 2)], didx.at[kb], sem_i).wait()

            @pl.when(k < CHUNKS_PER_SUB // 2 - 1)
            def _():
                pltpu.async_copy(dst_hbm.at[sid, pl.ds(c0 + 2, 2)],
                                 didx.at[1 - kb], sem_i)

            pltpu.async_copy(x2_hbm.at[sidx.at[c0 + 1]], rows_b, sem_b)
            pltpu.make_async_copy(x2_hbm.at[sidx.at[c0]], rows_a, sem_a).wait()
            pltpu.sync_copy(rows_a, acc.at[didx.at[kb, 0]], add=True)

            @pl.when(cid == 0)
            def _():
                count_phase(kb, 0, rows_a, sem_a)

            @pl.when(k < CHUNKS_PER_SUB // 2 - 1)
            def _():
                pltpu.async_copy(x2_hbm.at[sidx.at[c0 + 2]], rows_a, sem_a)

            pltpu.make_async_copy(
                x2_hbm.at[sidx.at[c0 + 1]], rows_b, sem_b).wait()
            pltpu.sync_copy(rows_b, acc.at[didx.at[kb, 1]], add=True)

            @pl.when(cid == 1)
            def _():
                count_phase(kb, 1, rows_b, sem_b)

            return 0

        lax.fori_loop(0, CHUNKS_PER_SUB // 2, step, 0)

        plsc.subcore_barrier()

        pltpu.sync_copy(acc.at[pl.ds(base, ROWS_PER_SUB)],
                        s_hbm.at[cid, pl.ds(base, ROWS_PER_SUB)])

        @pl.when(sid < CROWS // ZROWS)
        def _():
            pltpu.sync_copy(cnt.at[pl.ds(sid * ZROWS, ZROWS)],
                            cnt_hbm.at[cid, pl.ds(sid * ZROWS, ZROWS)])

    return sc_kernel(x2, srcs4, dst3, oh, zrs)


BR = 1000


def _tc_finish_body(x_ref, s0_ref, s1_ref, c0_ref, c1_ref, w_ref, b_ref,
                    o_ref):
    s = jnp.concatenate([s0_ref[0], s1_ref[0]], axis=1)
    c = jnp.maximum(c0_ref[...] + c1_ref[...], 1.0)
    h = s / c + x_ref[...]
    o_ref[...] = (
        jnp.dot(h, w_ref[...], preferred_element_type=jnp.float32,
                precision=lax.Precision.HIGHEST) + b_ref[...]
    )


def _tc_finish(x, s, c0, c1, W, b2):
    grid = (N_NODES // BR,)
    return pl.pallas_call(
        _tc_finish_body,
        grid=grid,
        in_specs=[
            pl.BlockSpec((BR, D_IN), lambda i: (i, 0)),
            pl.BlockSpec((1, BR, H), lambda i: (0, i, 0)),
            pl.BlockSpec((1, BR, H), lambda i: (1, i, 0)),
            pl.BlockSpec((BR, 1), lambda i: (i, 0)),
            pl.BlockSpec((BR, 1), lambda i: (i, 0)),
            pl.BlockSpec((D_IN, D_OUT), lambda i: (0, 0)),
            pl.BlockSpec((1, D_OUT), lambda i: (0, 0)),
        ],
        out_specs=pl.BlockSpec((BR, D_OUT), lambda i: (i, 0)),
        out_shape=jax.ShapeDtypeStruct((N_NODES, D_OUT), jnp.float32),
    )(x, s, s, c0, c1, W, b2)


def kernel(x, edge_index, W, b):
    src = edge_index[0].astype(jnp.int32)
    dst = edge_index[1].astype(jnp.int32)
    pad = E_PAD - N_EDGES
    srcs = jnp.concatenate([
        src, jnp.zeros((pad,), jnp.int32),
        src + N_NODES, jnp.full((pad,), N_NODES, jnp.int32),
    ]).reshape(NC, NS, CHUNKS_PER_SUB, EC)
    dst_p = jnp.concatenate(
        [dst, jnp.full((pad,), N_NODES, jnp.int32)]
    ).reshape(NS, CHUNKS_PER_SUB, EC)
    x2 = jnp.concatenate([x[:, :H], x[:, H:]], axis=0)
    oh = jnp.eye(128, dtype=jnp.float32)
    zrs = jnp.zeros((N_PAD, H), jnp.float32)

    s, cnt = _sc_segment_sum(x2, srcs, dst_p, oh, zrs)
    c0 = cnt[0].reshape(CROWS * 128, 1)[:N_NODES]
    c1 = cnt[1].reshape(CROWS * 128, 1)[:N_NODES]
    return _tc_finish(x, s, c0, c1, W, b.reshape(1, D_OUT))


# speedup vs baseline: 1.8986x; 1.0025x over previous
import functools

import jax
import jax.numpy as jnp
from jax import lax
from jax.experimental import pallas as pl
from jax.experimental.pallas import tpu as pltpu
from jax.experimental.pallas import tpu_sc as plsc

N_NODES = 10000
N_EDGES = 160000
D_IN = 256
D_OUT = 256
H = 128
NC = 2
NS = 16
EC = 128
CHUNKS_PER_SUB = 80
E_PAD = CHUNKS_PER_SUB * NS * EC
N_PAD = 10112
ROWS_PER_SUB = N_PAD // NS
ZROWS = 8
CROWS = 80


def _sc_segment_sum(x2, srcs4, dst3, oh, zrs):
    mesh = plsc.VectorSubcoreMesh(
        core_axis_name="c", subcore_axis_name="s",
        num_cores=NC, num_subcores=NS)

    @functools.partial(
        pl.kernel,
        mesh=mesh,
        out_type=(
            jax.ShapeDtypeStruct((NC, N_PAD, H), jnp.float32),
            jax.ShapeDtypeStruct((NC, CROWS, 128), jnp.float32),
        ),
        scratch_types=[
            pltpu.VMEM((CHUNKS_PER_SUB, EC), jnp.int32),
            pltpu.VMEM((2, 8, EC), jnp.int32),
            pltpu.VMEM((EC,), jnp.int32),
            pltpu.VMEM((EC,), jnp.int32),
            pltpu.VMEM((EC, H), jnp.float32),
            pltpu.VMEM((EC, H), jnp.float32),
            pltpu.VMEM_SHARED((N_PAD, H), jnp.float32),
            pltpu.VMEM_SHARED((CROWS, 128), jnp.float32),
            pltpu.VMEM_SHARED((128, 128), jnp.float32),
            pltpu.SemaphoreType.DMA,
            pltpu.SemaphoreType.DMA,
            pltpu.SemaphoreType.DMA,
        ],
    )
    def sc_kernel(x2_hbm, srcs_hbm, dst_hbm, oh_hbm, zrs_hbm, s_hbm,
                  cnt_hbm, sidx, didx, cl, ch, rows_a, rows_b, acc, cnt,
                  oh_sh, sem_a, sem_b, sem_i):
        cid = lax.axis_index("c")
        sid = lax.axis_index("s")
        base = sid * ROWS_PER_SUB
        pltpu.sync_copy(zrs_hbm.at[pl.ds(base, ROWS_PER_SUB)],
                        acc.at[pl.ds(base, ROWS_PER_SUB)])

        @pl.when(sid < CROWS // ZROWS)
        def _():
            pltpu.sync_copy(zrs_hbm.at[pl.ds(sid * ZROWS, ZROWS)],
                            cnt.at[pl.ds(sid * ZROWS, ZROWS)])

        pltpu.sync_copy(oh_hbm.at[pl.ds(sid * 8, 8)],
                        oh_sh.at[pl.ds(sid * 8, 8)])

        pltpu.sync_copy(srcs_hbm.at[cid, sid], sidx)
        pltpu.async_copy(dst_hbm.at[sid, pl.ds(0, 8)], didx.at[0], sem_i)

        plsc.subcore_barrier()

        def count_phase(kb, r, buf, sem):
            for j in range(EC // 16):
                d16 = didx[kb, r, pl.ds(j * 16, 16)]
                cl[pl.ds(j * 16, 16)] = lax.bitwise_and(d16, 127)
                ch[pl.ds(j * 16, 16)] = lax.shift_right_logical(d16, 7)
            pltpu.async_copy(oh_sh.at[cl], buf, sem).wait()
            pltpu.sync_copy(buf, cnt.at[ch], add=True)

        pltpu.async_copy(x2_hbm.at[sidx.at[0]], rows_a, sem_a)

        def step(k, _):
            c0 = 2 * k
            kb = lax.rem(lax.div(k, 4), 2)
            r0 = lax.rem(k, 4) * 2

            @pl.when(lax.rem(k, 4) == 0)
            def _():
                pltpu.make_async_copy(
                    dst_hbm.at[sid, pl.ds(0, 8)], didx.at[kb], sem_i).wait()

                @pl.when(k < CHUNKS_PER_SUB // 2 - 4)
                def _():
                    pltpu.async_copy(
                        dst_hbm.at[sid,
                                   pl.ds(pl.multiple_of(c0 + 8, 8), 8)],
                        didx.at[1 - kb], sem_i)

            pltpu.async_copy(x2_hbm.at[sidx.at[c0 + 1]], rows_b, sem_b)
            pltpu.make_async_copy(x2_hbm.at[sidx.at[c0]], rows_a, sem_a).wait()
            pltpu.sync_copy(rows_a, acc.at[didx.at[kb, r0]], add=True)

            @pl.when(cid == 0)
            def _():
                count_phase(kb, r0, rows_a, sem_a)

            @pl.when(k < CHUNKS_PER_SUB // 2 - 1)
            def _():
                pltpu.async_copy(x2_hbm.at[sidx.at[c0 + 2]], rows_a, sem_a)

            pltpu.make_async_copy(
                x2_hbm.at[sidx.at[c0 + 1]], rows_b, sem_b).wait()
            pltpu.sync_copy(rows_b, acc.at[didx.at[kb, r0 + 1]], add=True)

            @pl.when(cid == 1)
            def _():
                count_phase(kb, r0 + 1, rows_b, sem_b)

            return 0

        lax.fori_loop(0, CHUNKS_PER_SUB // 2, step, 0)

        plsc.subcore_barrier()

        pltpu.sync_copy(acc.at[pl.ds(base, ROWS_PER_SUB)],
                        s_hbm.at[cid, pl.ds(base, ROWS_PER_SUB)])

        @pl.when(sid < CROWS // ZROWS)
        def _():
            pltpu.sync_copy(cnt.at[pl.ds(sid * ZROWS, ZROWS)],
                            cnt_hbm.at[cid, pl.ds(sid * ZROWS, ZROWS)])

    return sc_kernel(x2, srcs4, dst3, oh, zrs)


BR = 1000


def _tc_finish_body(x_ref, s0_ref, s1_ref, c0_ref, c1_ref, w_ref, b_ref,
                    o_ref):
    s = jnp.concatenate([s0_ref[0], s1_ref[0]], axis=1)
    c = jnp.maximum(c0_ref[...] + c1_ref[...], 1.0)
    h = s / c + x_ref[...]
    o_ref[...] = (
        jnp.dot(h, w_ref[...], preferred_element_type=jnp.float32,
                precision=lax.Precision.HIGHEST) + b_ref[...]
    )


def _tc_finish(x, s, c0, c1, W, b2):
    grid = (N_NODES // BR,)
    return pl.pallas_call(
        _tc_finish_body,
        grid=grid,
        in_specs=[
            pl.BlockSpec((BR, D_IN), lambda i: (i, 0)),
            pl.BlockSpec((1, BR, H), lambda i: (0, i, 0)),
            pl.BlockSpec((1, BR, H), lambda i: (1, i, 0)),
            pl.BlockSpec((BR, 1), lambda i: (i, 0)),
            pl.BlockSpec((BR, 1), lambda i: (i, 0)),
            pl.BlockSpec((D_IN, D_OUT), lambda i: (0, 0)),
            pl.BlockSpec((1, D_OUT), lambda i: (0, 0)),
        ],
        out_specs=pl.BlockSpec((BR, D_OUT), lambda i: (i, 0)),
        out_shape=jax.ShapeDtypeStruct((N_NODES, D_OUT), jnp.float32),
    )(x, s, s, c0, c1, W, b2)


def kernel(x, edge_index, W, b):
    src = edge_index[0].astype(jnp.int32)
    dst = edge_index[1].astype(jnp.int32)
    pad = E_PAD - N_EDGES
    srcs = jnp.concatenate([
        src, jnp.zeros((pad,), jnp.int32),
        src + N_NODES, jnp.full((pad,), N_NODES, jnp.int32),
    ]).reshape(NC, NS, CHUNKS_PER_SUB, EC)
    dst_p = jnp.concatenate(
        [dst, jnp.full((pad,), N_NODES, jnp.int32)]
    ).reshape(NS, CHUNKS_PER_SUB, EC)
    x2 = jnp.concatenate([x[:, :H], x[:, H:]], axis=0)
    oh = jnp.eye(128, dtype=jnp.float32)
    zrs = jnp.zeros((N_PAD, H), jnp.float32)

    s, cnt = _sc_segment_sum(x2, srcs, dst_p, oh, zrs)
    c0 = cnt[0].reshape(CROWS * 128, 1)[:N_NODES]
    c1 = cnt[1].reshape(CROWS * 128, 1)[:N_NODES]
    return _tc_finish(x, s, c0, c1, W, b.reshape(1, D_OUT))


# speedup vs baseline: 1.9055x; 1.0037x over previous
import functools

import jax
import jax.numpy as jnp
from jax import lax
from jax.experimental import pallas as pl
from jax.experimental.pallas import tpu as pltpu
from jax.experimental.pallas import tpu_sc as plsc

N_NODES = 10000
N_EDGES = 160000
D_IN = 256
D_OUT = 256
H = 128
NC = 2
NS = 16
EC = 128
CHUNKS_PER_SUB = 80
E_PAD = CHUNKS_PER_SUB * NS * EC
N_PAD = 10112
ROWS_PER_SUB = N_PAD // NS
ZROWS = 8
CROWS = 80


def _sc_segment_sum(x2, srcs4, dst3, oh, zrs):
    mesh = plsc.VectorSubcoreMesh(
        core_axis_name="c", subcore_axis_name="s",
        num_cores=NC, num_subcores=NS)

    @functools.partial(
        pl.kernel,
        mesh=mesh,
        out_type=(
            jax.ShapeDtypeStruct((NC, N_PAD, H), jnp.float32),
            jax.ShapeDtypeStruct((NC, CROWS, 128), jnp.float32),
        ),
        scratch_types=[
            pltpu.VMEM((CHUNKS_PER_SUB, EC), jnp.int32),
            pltpu.VMEM((2, 8, EC), jnp.int32),
            pltpu.VMEM((EC,), jnp.int32),
            pltpu.VMEM((EC,), jnp.int32),
            pltpu.VMEM((EC, H), jnp.float32),
            pltpu.VMEM((EC, H), jnp.float32),
            pltpu.VMEM_SHARED((N_PAD, H), jnp.float32),
            pltpu.VMEM_SHARED((CROWS, 128), jnp.float32),
            pltpu.VMEM_SHARED((128, 128), jnp.float32),
            pltpu.SemaphoreType.DMA,
            pltpu.SemaphoreType.DMA,
            pltpu.SemaphoreType.DMA,
        ],
    )
    def sc_kernel(x2_hbm, srcs_hbm, dst_hbm, oh_hbm, zrs_hbm, s_hbm,
                  cnt_hbm, sidx, didx, cl, ch, rows_a, rows_b, acc, cnt,
                  oh_sh, sem_a, sem_b, sem_i):
        cid = lax.axis_index("c")
        sid = lax.axis_index("s")
        base = sid * ROWS_PER_SUB
        pltpu.sync_copy(zrs_hbm.at[pl.ds(base, ROWS_PER_SUB)],
                        acc.at[pl.ds(base, ROWS_PER_SUB)])

        @pl.when(sid < CROWS // ZROWS)
        def _():
            pltpu.sync_copy(zrs_hbm.at[pl.ds(sid * ZROWS, ZROWS)],
                            cnt.at[pl.ds(sid * ZROWS, ZROWS)])

        pltpu.sync_copy(oh_hbm.at[pl.ds(sid * 8, 8)],
                        oh_sh.at[pl.ds(sid * 8, 8)])

        pltpu.sync_copy(srcs_hbm.at[cid, sid], sidx)
        pltpu.async_copy(dst_hbm.at[sid, pl.ds(0, 8)], didx.at[0], sem_i)

        plsc.subcore_barrier()

        def count_phase(kb, r, buf, sem):
            for j in range(EC // 16):
                d16 = didx[kb, r, pl.ds(j * 16, 16)]
                cl[pl.ds(j * 16, 16)] = lax.bitwise_and(d16, 127)
                ch[pl.ds(j * 16, 16)] = lax.shift_right_logical(d16, 7)
            pltpu.async_copy(oh_sh.at[cl], buf, sem).wait()
            pltpu.sync_copy(buf, cnt.at[ch], add=True)

        pltpu.async_copy(x2_hbm.at[sidx.at[0]], rows_a, sem_a)

        def step(k, _):
            c0 = 2 * k
            kb = lax.rem(lax.div(k, 4), 2)
            r0 = lax.rem(k, 4) * 2

            @pl.when(lax.rem(k, 4) == 0)
            def _():
                pltpu.make_async_copy(
                    dst_hbm.at[sid, pl.ds(0, 8)], didx.at[kb], sem_i).wait()

                @pl.when(k < CHUNKS_PER_SUB // 2 - 4)
                def _():
                    pltpu.async_copy(
                        dst_hbm.at[sid,
                                   pl.ds(pl.multiple_of(c0 + 8, 8), 8)],
                        didx.at[1 - kb], sem_i)

            pltpu.async_copy(x2_hbm.at[sidx.at[c0 + 1]], rows_b, sem_b)
            pltpu.make_async_copy(x2_hbm.at[sidx.at[c0]], rows_a, sem_a).wait()
            pltpu.sync_copy(rows_a, acc.at[didx.at[kb, r0]], add=True)

            @pl.when(k < CHUNKS_PER_SUB // 2 - 1)
            def _():
                pltpu.async_copy(x2_hbm.at[sidx.at[c0 + 2]], rows_a, sem_a)

            pltpu.make_async_copy(
                x2_hbm.at[sidx.at[c0 + 1]], rows_b, sem_b).wait()
            pltpu.sync_copy(rows_b, acc.at[didx.at[kb, r0 + 1]], add=True)

            @pl.when(cid == 0)
            def _():
                count_phase(kb, r0, rows_b, sem_b)

            @pl.when(cid == 1)
            def _():
                count_phase(kb, r0 + 1, rows_b, sem_b)

            return 0

        lax.fori_loop(0, CHUNKS_PER_SUB // 2, step, 0)

        plsc.subcore_barrier()

        pltpu.sync_copy(acc.at[pl.ds(base, ROWS_PER_SUB)],
                        s_hbm.at[cid, pl.ds(base, ROWS_PER_SUB)])

        @pl.when(sid < CROWS // ZROWS)
        def _():
            pltpu.sync_copy(cnt.at[pl.ds(sid * ZROWS, ZROWS)],
                            cnt_hbm.at[cid, pl.ds(sid * ZROWS, ZROWS)])

    return sc_kernel(x2, srcs4, dst3, oh, zrs)


BR = 1000


def _tc_finish_body(x_ref, s0_ref, s1_ref, c0_ref, c1_ref, w_ref, b_ref,
                    o_ref):
    s = jnp.concatenate([s0_ref[0], s1_ref[0]], axis=1)
    c = jnp.maximum(c0_ref[...] + c1_ref[...], 1.0)
    h = s / c + x_ref[...]
    o_ref[...] = (
        jnp.dot(h, w_ref[...], preferred_element_type=jnp.float32,
                precision=lax.Precision.HIGHEST) + b_ref[...]
    )


def _tc_finish(x, s, c0, c1, W, b2):
    grid = (N_NODES // BR,)
    return pl.pallas_call(
        _tc_finish_body,
        grid=grid,
        in_specs=[
            pl.BlockSpec((BR, D_IN), lambda i: (i, 0)),
            pl.BlockSpec((1, BR, H), lambda i: (0, i, 0)),
            pl.BlockSpec((1, BR, H), lambda i: (1, i, 0)),
            pl.BlockSpec((BR, 1), lambda i: (i, 0)),
            pl.BlockSpec((BR, 1), lambda i: (i, 0)),
            pl.BlockSpec((D_IN, D_OUT), lambda i: (0, 0)),
            pl.BlockSpec((1, D_OUT), lambda i: (0, 0)),
        ],
        out_specs=pl.BlockSpec((BR, D_OUT), lambda i: (i, 0)),
        out_shape=jax.ShapeDtypeStruct((N_NODES, D_OUT), jnp.float32),
    )(x, s, s, c0, c1, W, b2)


def kernel(x, edge_index, W, b):
    src = edge_index[0].astype(jnp.int32)
    dst = edge_index[1].astype(jnp.int32)
    pad = E_PAD - N_EDGES
    srcs = jnp.concatenate([
        src, jnp.zeros((pad,), jnp.int32),
        src + N_NODES, jnp.full((pad,), N_NODES, jnp.int32),
    ]).reshape(NC, NS, CHUNKS_PER_SUB, EC)
    dst_p = jnp.concatenate(
        [dst, jnp.full((pad,), N_NODES, jnp.int32)]
    ).reshape(NS, CHUNKS_PER_SUB, EC)
    x2 = jnp.concatenate([x[:, :H], x[:, H:]], axis=0)
    oh = jnp.eye(128, dtype=jnp.float32)
    zrs = jnp.zeros((N_PAD, H), jnp.float32)

    s, cnt = _sc_segment_sum(x2, srcs, dst_p, oh, zrs)
    c0 = cnt[0].reshape(CROWS * 128, 1)[:N_NODES]
    c1 = cnt[1].reshape(CROWS * 128, 1)[:N_NODES]
    return _tc_finish(x, s, c0, c1, W, b.reshape(1, D_OUT))
